# baseline (speedup 1.0000x reference)
import jax
import jax.numpy as jnp
from jax import lax
from jax.experimental import pallas as pl
from jax.experimental.pallas import tpu as pltpu
from jax.experimental.pallas import tpu_sc as plsc

B = 1024
S = 200
H = 128
HA = 104
HB = S - HA
CP = 112
NC = 2
NS = 16
NW = NC * NS
ROWS_PER_W = B // NW
NH = 2 * ROWS_PER_W
LANES = 16
KCH = H // LANES
NSLOT = 4


def _body(ids_h, cidx_h, pos_h, typ_h, ten_h, word_h, out_h,
          pos_v, combo_v, typ_v, ids_v, cidx_v,
          acc0, acc1, acc2, acc3,
          g0, g1, g2, g3, w0, w1, w2, w3):
    c = lax.axis_index("c")
    s = lax.axis_index("s")
    wid = s * NC + c
    accs = (acc0, acc1, acc2, acc3)
    gsems = (g0, g1, g2, g3)
    wsems = (w0, w1, w2, w3)

    pltpu.sync_copy(ids_h.at[pl.ds(wid * NH * HA, NH * HA)], ids_v)
    pltpu.sync_copy(cidx_h.at[pl.ds(wid * NH * CP, NH * CP)], cidx_v)
    pltpu.sync_copy(pos_h, pos_v)
    pltpu.sync_copy(typ_h, typ_v)
    pltpu.sync_copy(ten_h, combo_v.at[pl.ds(0, HA)])
    pltpu.sync_copy(ten_h, combo_v.at[pl.ds(100, HA)])

    def build(t, carry):
        for half in range(2):
            for k in range(KCH):
                sl = pl.ds(k * LANES, LANES)
                combo_v[half * 100 + t, sl] = (
                    combo_v[half * 100 + t, sl]
                    + typ_v[pl.ds(half * H + k * LANES, LANES)])
        return carry
    lax.fori_loop(0, 100, build, 0)

    def issue_gather(h, slot):
        return pltpu.async_copy(
            word_h.at[ids_v.at[pl.ds(h * HA, HA)]], accs[slot], gsems[slot])

    issue_gather(0, 0)
    issue_gather(1, 1)

    def do_token(acc, t, s_off, ct):
        for k in range(KCH):
            sl = pl.ds(k * LANES, LANES)
            acc[t, sl] = acc[t, sl] + pos_v[s_off + t, sl] + combo_v[ct, sl]

    def step(p, carry):
        for j in range(NSLOT):
            h = p * NSLOT + j
            acc = accs[j]
            even = (j % 2 == 0)
            nvalid = HA if even else HB
            s_off = 0 if even else HA
            b = wid * ROWS_PER_W + (h // 2)
            cb = h * CP

            pltpu.make_async_copy(
                word_h.at[ids_v.at[pl.ds(h * HA, HA)]], acc,
                gsems[j]).wait()

            def group(q, inner):
                t0 = q * LANES
                chunk = cidx_v[pl.ds(cb + t0, LANES)]
                for i in range(LANES):
                    do_token(acc, t0 + i, s_off, chunk[i])
                return inner
            lax.fori_loop(0, nvalid // LANES, group, 0)
            if nvalid % LANES:
                t0 = (nvalid // LANES) * LANES
                chunk = cidx_v[pl.ds(cb + t0, LANES)]
                for i in range(nvalid % LANES):
                    do_token(acc, t0 + i, s_off, chunk[i])

            wb = pltpu.async_copy(
                acc.at[pl.ds(0, nvalid)],
                out_h.at[b, pl.ds(s_off, nvalid)], wsems[j])

            jn = (j + 2) % NSLOT
            n_even = (jn % 2 == 0)
            n_valid = HA if n_even else HB
            n_soff = 0 if n_even else HA

            @pl.when(jnp.logical_and(h >= 2, h < NH - 2))
            def _():
                bp = wid * ROWS_PER_W + ((h - 2) // 2)
                pltpu.make_async_copy(
                    accs[jn].at[pl.ds(0, n_valid)],
                    out_h.at[bp, pl.ds(n_soff, n_valid)], wsems[jn]).wait()

            @pl.when(h < NH - 2)
            def _():
                issue_gather(h + 2, jn)
        return carry
    lax.fori_loop(0, NH // NSLOT, step, 0)

    for j in range(NSLOT):
        h = NH - NSLOT + j
        even = (j % 2 == 0)
        nvalid = HA if even else HB
        s_off = 0 if even else HA
        b = wid * ROWS_PER_W + (h // 2)
        pltpu.make_async_copy(
            accs[j].at[pl.ds(0, nvalid)],
            out_h.at[b, pl.ds(s_off, nvalid)], wsems[j]).wait()


@jax.jit
def _run(ids, cidx, pos, typ, ten, word):
    mesh = plsc.VectorSubcoreMesh(core_axis_name="c", subcore_axis_name="s")
    return pl.kernel(
        _body,
        out_type=jax.ShapeDtypeStruct((B, S, H), jnp.float32),
        mesh=mesh,
        scratch_types=[
            pltpu.VMEM((S, H), jnp.float32),
            pltpu.VMEM((2 * HA, H), jnp.float32),
            pltpu.VMEM((2 * H,), jnp.float32),
            pltpu.VMEM((NH * HA,), jnp.int32),
            pltpu.VMEM((NH * CP,), jnp.int32),
            pltpu.VMEM((HA, H), jnp.float32),
            pltpu.VMEM((HA, H), jnp.float32),
            pltpu.VMEM((HA, H), jnp.float32),
            pltpu.VMEM((HA, H), jnp.float32),
            pltpu.SemaphoreType.DMA,
            pltpu.SemaphoreType.DMA,
            pltpu.SemaphoreType.DMA,
            pltpu.SemaphoreType.DMA,
            pltpu.SemaphoreType.DMA,
            pltpu.SemaphoreType.DMA,
            pltpu.SemaphoreType.DMA,
            pltpu.SemaphoreType.DMA,
        ],
    )(ids, cidx, pos, typ, ten, word)


def kernel(input_ids, token_type_ids, tenant_ids, W_word, W_pos, W_type, W_tenant):
    ids = input_ids.astype(jnp.int32)
    cidx = (token_type_ids.astype(jnp.int32) * 100
            + tenant_ids.astype(jnp.int32))
    ids_p = jnp.zeros((B, 2, HA), jnp.int32)
    ids_p = ids_p.at[:, 0, :].set(ids[:, :HA])
    ids_p = ids_p.at[:, 1, :HB].set(ids[:, HA:])
    cidx_p = jnp.zeros((B, 2, CP), jnp.int32)
    cidx_p = cidx_p.at[:, 0, :HA].set(cidx[:, :HA])
    cidx_p = cidx_p.at[:, 1, :HB].set(cidx[:, HA:])
    pos = W_pos[:S]
    ten = jnp.pad(W_tenant, ((0, HA - W_tenant.shape[0]), (0, 0)))
    return _run(ids_p.reshape(-1), cidx_p.reshape(-1), pos,
                W_type.reshape(-1), ten, W_word)


# speedup vs baseline: 1.0124x; 1.0124x over previous
import jax
import jax.numpy as jnp
from jax import lax
from jax.experimental import pallas as pl
from jax.experimental.pallas import tpu as pltpu
from jax.experimental.pallas import tpu_sc as plsc

B = 1024
S = 200
H = 128
HA = 104
HB = S - HA
NC = 2
NS = 16
NW = NC * NS
ROWS_PER_W = B // NW
NH = 2 * ROWS_PER_W
LANES = 16
KCH = H // LANES
NSLOT = 4


def _body(ids_h, cidx_h, pos_h, typ_h, ten_h, word_h, out_h,
          pos_v, typ_v, ids_v, cidx_v,
          acc0, acc1, acc2, acc3, cmb0, cmb1, combo_sh,
          g0, g1, g2, g3, w0, w1, w2, w3, c0, c1):
    c = lax.axis_index("c")
    s = lax.axis_index("s")
    wid = s * NC + c
    accs = (acc0, acc1, acc2, acc3)
    cmbs = (cmb0, cmb1)
    gsems = (g0, g1, g2, g3)
    wsems = (w0, w1, w2, w3)
    csems = (c0, c1)

    @pl.when(s == 0)
    def _():
        pltpu.sync_copy(ten_h, acc1)
        pltpu.sync_copy(typ_h, typ_v)

        def build_a(t, carry):
            for k in range(KCH):
                sl = pl.ds(k * LANES, LANES)
                acc0[t, sl] = acc1[t, sl] + typ_v[pl.ds(k * LANES, LANES)]
            return carry
        lax.fori_loop(0, 100, build_a, 0)
        for t in range(100, HA):
            for k in range(KCH):
                sl = pl.ds(k * LANES, LANES)
                acc0[t, sl] = (acc1[t - 100, sl]
                               + typ_v[pl.ds(H + k * LANES, LANES)])
        pltpu.sync_copy(acc0, combo_sh.at[pl.ds(0, HA)])

        def build_b(t, carry):
            for k in range(KCH):
                sl = pl.ds(k * LANES, LANES)
                acc0[t, sl] = (acc1[t + 4, sl]
                               + typ_v[pl.ds(H + k * LANES, LANES)])
            return carry
        lax.fori_loop(0, HB, build_b, 0)
        pltpu.sync_copy(acc0.at[pl.ds(0, HB)], combo_sh.at[pl.ds(HA, HB)])

    plsc.subcore_barrier()

    pltpu.sync_copy(ids_h.at[pl.ds(wid * NH * HA, NH * HA)], ids_v)
    pltpu.sync_copy(cidx_h.at[pl.ds(wid * NH * HA, NH * HA)], cidx_v)
    pltpu.sync_copy(pos_h, pos_v)

    def issue_word(h, slot):
        pltpu.async_copy(
            word_h.at[ids_v.at[pl.ds(h * HA, HA)]], accs[slot], gsems[slot])

    def wait_word(h, slot):
        pltpu.make_async_copy(
            word_h.at[ids_v.at[pl.ds(h * HA, HA)]], accs[slot],
            gsems[slot]).wait()

    def issue_combo(h, slot):
        pltpu.async_copy(
            combo_sh.at[cidx_v.at[pl.ds(h * HA, HA)]], cmbs[slot],
            csems[slot])

    def wait_combo(h, slot):
        pltpu.make_async_copy(
            combo_sh.at[cidx_v.at[pl.ds(h * HA, HA)]], cmbs[slot],
            csems[slot]).wait()

    issue_word(0, 0)
    issue_word(1, 1)
    issue_combo(0, 0)

    def step(p, carry):
        for j in range(NSLOT):
            h = p * NSLOT + j
            acc = accs[j]
            cmb = cmbs[j % 2]
            even = (j % 2 == 0)
            nvalid = HA if even else HB
            s_off = 0 if even else HA
            b = wid * ROWS_PER_W + (h // 2)

            wait_word(h, j)
            wait_combo(h, j % 2)

            @pl.when(h < NH - 1)
            def _():
                issue_combo(h + 1, (j + 1) % 2)

            @plsc.parallel_loop(0, nvalid)
            def tok(t):
                for k in range(KCH):
                    sl = pl.ds(k * LANES, LANES)
                    acc[t, sl] = (acc[t, sl] + cmb[t, sl]
                                  + pos_v[s_off + t, sl])

            pltpu.async_copy(
                acc.at[pl.ds(0, nvalid)],
                out_h.at[b, pl.ds(s_off, nvalid)], wsems[j])

            jn = (j + 2) % NSLOT
            n_even = (jn % 2 == 0)
            n_valid = HA if n_even else HB
            n_soff = 0 if n_even else HA

            @pl.when(jnp.logical_and(h >= 2, h < NH - 2))
            def _():
                bp = wid * ROWS_PER_W + ((h - 2) // 2)
                pltpu.make_async_copy(
                    accs[jn].at[pl.ds(0, n_valid)],
                    out_h.at[bp, pl.ds(n_soff, n_valid)], wsems[jn]).wait()

            @pl.when(h < NH - 2)
            def _():
                issue_word(h + 2, jn)
        return carry
    lax.fori_loop(0, NH // NSLOT, step, 0)

    for j in range(NSLOT):
        h = NH - NSLOT + j
        even = (j % 2 == 0)
        nvalid = HA if even else HB
        s_off = 0 if even else HA
        b = wid * ROWS_PER_W + (h // 2)
        pltpu.make_async_copy(
            accs[j].at[pl.ds(0, nvalid)],
            out_h.at[b, pl.ds(s_off, nvalid)], wsems[j]).wait()


@jax.jit
def _run(ids, cidx, pos, typ, ten, word):
    mesh = plsc.VectorSubcoreMesh(core_axis_name="c", subcore_axis_name="s")
    return pl.kernel(
        _body,
        out_type=jax.ShapeDtypeStruct((B, S, H), jnp.float32),
        mesh=mesh,
        scratch_types=[
            pltpu.VMEM((S, H), jnp.float32),
            pltpu.VMEM((2 * H,), jnp.float32),
            pltpu.VMEM((NH * HA,), jnp.int32),
            pltpu.VMEM((NH * HA,), jnp.int32),
            pltpu.VMEM((HA, H), jnp.float32),
            pltpu.VMEM((HA, H), jnp.float32),
            pltpu.VMEM((HA, H), jnp.float32),
            pltpu.VMEM((HA, H), jnp.float32),
            pltpu.VMEM((HA, H), jnp.float32),
            pltpu.VMEM((HA, H), jnp.float32),
            pltpu.VMEM_SHARED((2 * HA, H), jnp.float32),
            pltpu.SemaphoreType.DMA,
            pltpu.SemaphoreType.DMA,
            pltpu.SemaphoreType.DMA,
            pltpu.SemaphoreType.DMA,
            pltpu.SemaphoreType.DMA,
            pltpu.SemaphoreType.DMA,
            pltpu.SemaphoreType.DMA,
            pltpu.SemaphoreType.DMA,
            pltpu.SemaphoreType.DMA,
            pltpu.SemaphoreType.DMA,
        ],
    )(ids, cidx, pos, typ, ten, word)


def kernel(input_ids, token_type_ids, tenant_ids, W_word, W_pos, W_type, W_tenant):
    ids = input_ids.astype(jnp.int32)
    cidx = (token_type_ids.astype(jnp.int32) * 100
            + tenant_ids.astype(jnp.int32))
    ids_p = jnp.zeros((B, 2, HA), jnp.int32)
    ids_p = ids_p.at[:, 0, :].set(ids[:, :HA])
    ids_p = ids_p.at[:, 1, :HB].set(ids[:, HA:])
    cidx_p = jnp.zeros((B, 2, HA), jnp.int32)
    cidx_p = cidx_p.at[:, 0, :].set(cidx[:, :HA])
    cidx_p = cidx_p.at[:, 1, :HB].set(cidx[:, HA:])
    pos = W_pos[:S]
    ten = jnp.pad(W_tenant, ((0, HA - W_tenant.shape[0]), (0, 0)))
    return _run(ids_p.reshape(-1), cidx_p.reshape(-1), pos,
                W_type.reshape(-1), ten, W_word)
